# baseline (speedup 1.0000x reference)
import functools

import jax
import jax.numpy as jnp
from jax import lax
from jax.experimental import pallas as pl
from jax.experimental.pallas import tpu as pltpu
from jax.experimental.pallas import tpu_sc as plsc

VEC = 64
B = 4096
CTX = 20
NSAMP = 10
NC, NS = 2, 16
NW = NC * NS
BPW = B // NW
LANES = 16
NV = VEC // LANES


_DW = 4


def _doc_body(pt_hbm, doc_f, out_hbm, doc_idx, slab0, slab1, docbuf, sem):
    wid = lax.axis_index("s") * NC + lax.axis_index("c")
    pltpu.sync_copy(doc_f.at[pl.ds(wid * BPW, BPW)], doc_idx)

    lane = lax.iota(jnp.int32, LANES)
    nwaves = BPW // _DW
    slabs = [slab0, slab1]
    nbuf = len(slabs)
    pend = [None] * nbuf

    def fire(w):
        jv = doc_idx[pl.ds((w * _DW // LANES) * LANES, LANES)]
        buf = slabs[w % nbuf]
        cps = []
        for t in range(_DW):
            off = pl.multiple_of(
                (jv[(w * _DW + t) % LANES] >> 7) << 7, 128)
            cps.append(pltpu.async_copy(pt_hbm.at[:, pl.ds(off, 128)],
                                        buf.at[:, pl.ds(t * 128, 128)], sem))
        return cps

    def extract(w):
        jv = doc_idx[pl.ds((w * _DW // LANES) * LANES, LANES)]
        buf = slabs[w % nbuf]
        for t in range(_DW):
            sid = jv[(w * _DW + t) % LANES]
            col = jnp.full((LANES,), 0, jnp.int32) + (t * 128 + (sid & 127))
            dstrow = jnp.full((LANES,), w * _DW + t, jnp.int32)
            for k in range(NV):
                rows = k * LANES + lane
                v = plsc.load_gather(buf, [rows, col])
                plsc.store_scatter(docbuf, [dstrow, rows], v)

    pend[0] = fire(0)
    for w in range(nwaves):
        if w + 1 < nwaves:
            pend[(w + 1) % nbuf] = fire(w + 1)
        for cp in pend[w % nbuf]:
            cp.wait()
        extract(w)

    pltpu.sync_copy(docbuf, out_hbm.at[wid])


@functools.partial(
    pl.kernel,
    out_type=jax.ShapeDtypeStruct((NW, BPW, VEC), jnp.float32),
    mesh=plsc.VectorSubcoreMesh(core_axis_name="c", subcore_axis_name="s"),
    scratch_types=[
        pltpu.VMEM((BPW,), jnp.int32),
        pltpu.VMEM((VEC, _DW * 128), jnp.float32),
        pltpu.VMEM((VEC, _DW * 128), jnp.float32),
        pltpu.VMEM((BPW, VEC), jnp.float32),
        pltpu.SemaphoreType.DMA,
    ],
    compiler_params=pltpu.CompilerParams(needs_layout_passes=False,
                                         use_tc_tiling_on_sc=True),
)
def _doc_kernel(pt_hbm, doc_f, out_hbm, *scratch):
    _doc_body(pt_hbm, doc_f, out_hbm, *scratch)


_CTX_CHUNK = 64
_N_CTX_CHUNKS = BPW // _CTX_CHUNK
_CTX_ROWS = _CTX_CHUNK * CTX


def _sc_body(ctx_w, samp_f, docrows_hbm, w_hbm, ot_hbm, out_hbm,
             ctx_idx, samp_idx, big, inp, res, sem):
    wid = lax.axis_index("s") * NC + lax.axis_index("c")

    pltpu.sync_copy(ctx_w.at[pl.ds(wid * (CTX * BPW), CTX * BPW)],
                    ctx_idx)
    pltpu.sync_copy(samp_f.at[pl.ds(wid * (NSAMP * BPW), NSAMP * BPW)],
                    samp_idx)

    pltpu.sync_copy(docrows_hbm.at[wid], inp)
    cps = [
        pltpu.async_copy(w_hbm.at[ctx_idx.at[pl.ds(c * BPW, BPW)]],
                         inp, sem, add=True)
        for c in range(CTX)
    ]
    cps += [
        pltpu.async_copy(ot_hbm.at[samp_idx.at[pl.ds(j * 128, 128)]],
                         big.at[pl.ds(j * 128, 128)], sem)
        for j in range(10)
    ]
    for cp in cps:
        cp.wait()

    def score_body(i, _):
        lane = lax.iota(jnp.int32, LANES)
        smask = lane < NSAMP
        rows = i * NSAMP + lane
        vin = [inp[i, pl.ds(k * LANES, LANES)] for k in range(NV)]
        acc = jnp.zeros((LANES,), jnp.float32)
        for d in range(VEC):
            col = jnp.full((LANES,), d, jnp.int32)
            g = plsc.load_gather(big, [rows, col], mask=smask)
            acc = acc + vin[d // LANES][d % LANES] * g
        plsc.store_compressed(res.at[pl.ds(i * NSAMP, LANES)], acc, mask=smask)
        return 0

    lax.fori_loop(0, BPW, score_body, 0)

    pltpu.sync_copy(res.at[pl.ds(0, BPW * NSAMP)],
                    out_hbm.at[pl.ds(wid * (BPW * NSAMP), BPW * NSAMP)])


@functools.partial(
    pl.kernel,
    out_type=jax.ShapeDtypeStruct((B * NSAMP,), jnp.float32),
    mesh=plsc.VectorSubcoreMesh(core_axis_name="c", subcore_axis_name="s"),
    scratch_types=[
        pltpu.VMEM((CTX * BPW,), jnp.int32),
        pltpu.VMEM((NSAMP * BPW,), jnp.int32),
        pltpu.VMEM((NSAMP * BPW + 8, VEC), jnp.float32),
        pltpu.VMEM((BPW, VEC), jnp.float32),
        pltpu.VMEM((BPW * NSAMP + 8, ), jnp.float32),
        pltpu.SemaphoreType.DMA,
    ],
    compiler_params=pltpu.CompilerParams(needs_layout_passes=False,
                                         use_tc_tiling_on_sc=False),
)
def _sc_kernel(ctx_w, samp_f, docrows_hbm, w_hbm, ot_hbm, out_hbm, *scratch):
    _sc_body(ctx_w, samp_f, docrows_hbm, w_hbm, ot_hbm, out_hbm, *scratch)


def kernel(doc_ids, context_ids, sample_ids, paragraph_matrix, word_matrix,
           outputs):
    ot = outputs.T
    doc_f = doc_ids.astype(jnp.int32).reshape(B)
    ctx_w = (context_ids.astype(jnp.int32)
             .reshape(NW, BPW, CTX).transpose(0, 2, 1).reshape(B * CTX))
    samp_f = sample_ids.astype(jnp.int32).reshape(B * NSAMP)
    pt = paragraph_matrix.T
    docrows = _doc_kernel(pt, doc_f)
    res = _sc_kernel(ctx_w, samp_f, docrows, word_matrix, ot)
    return res.reshape(B, NSAMP)
